# baseline (speedup 1.0000x reference)
import functools

import jax
import jax.numpy as jnp
from jax import lax
from jax.experimental import pallas as pl
from jax.experimental.pallas import tpu as pltpu, tpu_sc as plsc

TOPK = 8
NUM_EXPERTS = 64
NUM_TOKENS = 32768
LANES = 16


def _make_sc_kernel():
    info = plsc.get_sparse_core_info()
    nc, ns = info.num_cores, info.num_subcores
    nw = nc * ns
    assert NUM_TOKENS % nw == 0
    tok_per_w = NUM_TOKENS // nw

    mesh = plsc.VectorSubcoreMesh(core_axis_name="c", subcore_axis_name="s")

    @functools.partial(
        pl.kernel,
        out_type=(
            jax.ShapeDtypeStruct((NUM_TOKENS, TOPK), jnp.float32),
            jax.ShapeDtypeStruct((NUM_TOKENS, TOPK), jnp.int32),
        ),
        mesh=mesh,
        compiler_params=pltpu.CompilerParams(needs_layout_passes=False,
                                             use_tc_tiling_on_sc=False),
        scratch_types=[
            pltpu.VMEM((tok_per_w, NUM_EXPERTS), jnp.float32),
            pltpu.VMEM((tok_per_w, TOPK), jnp.float32),
            pltpu.VMEM((tok_per_w, TOPK), jnp.int32),
        ],
    )
    def sc_kernel(logits_hbm, out_w_hbm, out_i_hbm, lbuf, wbuf, ibuf):
        wid = lax.axis_index("c") * ns + lax.axis_index("s")
        base = wid * tok_per_w

        pltpu.sync_copy(logits_hbm.at[pl.ds(base, tok_per_w)], lbuf)

        iota = lax.iota(jnp.int32, LANES)
        lane_lo = iota < TOPK

        def topk_one(tok):
            ks, vs = [], []
            for g in range(NUM_EXPERTS // LANES):
                x = lbuf[tok, pl.ds(g * LANES, LANES)]
                k_, v_ = plsc.sort_key_val(x, iota + g * LANES,
                                           descending=(g % 2 == 0))
                ks.append(k_)
                vs.append(v_)
            p = jnp.where(lane_lo, ks[0], ks[1])
            pi = jnp.where(lane_lo, vs[0], vs[1])
            q = jnp.where(lane_lo, ks[2], ks[3])
            qi = jnp.where(lane_lo, vs[2], vs[3])
            p, pi = plsc.sort_key_val(p, pi, descending=True)
            q, qi = plsc.sort_key_val(q, qi, descending=False)
            r = jnp.where(lane_lo, p, q)
            ri = jnp.where(lane_lo, pi, qi)
            r, ri = plsc.sort_key_val(r, ri, descending=True)
            e = jnp.where(lane_lo, jnp.exp(r), 0.0)
            w = e / jnp.broadcast_to(jnp.sum(e), (LANES,))
            return w, ri

        @plsc.parallel_loop(0, tok_per_w, unroll=8)
        def body(tok):
            w, ri = topk_one(tok)
            rows = jnp.full((LANES,), tok, jnp.int32)
            plsc.store_scatter(wbuf, [rows, iota], w, mask=lane_lo)
            plsc.store_scatter(ibuf, [rows, iota], ri, mask=lane_lo)

        pltpu.sync_copy(wbuf, out_w_hbm.at[pl.ds(base, tok_per_w)])
        pltpu.sync_copy(ibuf, out_i_hbm.at[pl.ds(base, tok_per_w)])

    return sc_kernel


_SC_KERNEL = _make_sc_kernel()


def kernel(router_logits_fp32, topk_ids, topk_weights):
    w, ids = _SC_KERNEL(router_logits_fp32)
    return (w.astype(topk_weights.dtype), ids.astype(topk_ids.dtype))


# speedup vs baseline: 1.0029x; 1.0029x over previous
import functools

import jax
import jax.numpy as jnp
from jax import lax
from jax.experimental import pallas as pl
from jax.experimental.pallas import tpu as pltpu, tpu_sc as plsc

TOPK = 8
NUM_EXPERTS = 64
NUM_TOKENS = 32768
LANES = 16


def _make_sc_kernel():
    info = plsc.get_sparse_core_info()
    nc, ns = info.num_cores, info.num_subcores
    nw = nc * ns
    assert NUM_TOKENS % nw == 0
    tok_per_w = NUM_TOKENS // nw

    mesh = plsc.VectorSubcoreMesh(core_axis_name="c", subcore_axis_name="s")

    @functools.partial(
        pl.kernel,
        out_type=(
            jax.ShapeDtypeStruct((NUM_TOKENS * TOPK // 128, 128), jnp.float32),
            jax.ShapeDtypeStruct((NUM_TOKENS * TOPK // 128, 128), jnp.int32),
        ),
        mesh=mesh,
        compiler_params=pltpu.CompilerParams(needs_layout_passes=False,
                                             use_tc_tiling_on_sc=False),
        scratch_types=[
            pltpu.VMEM((tok_per_w, NUM_EXPERTS), jnp.float32),
            pltpu.VMEM((tok_per_w * TOPK // 128, 128), jnp.float32),
            pltpu.VMEM((tok_per_w * TOPK // 128, 128), jnp.int32),
        ],
    )
    def sc_kernel(logits_hbm, out_w_hbm, out_i_hbm, lbuf, wbuf, ibuf):
        wid = lax.axis_index("c") * ns + lax.axis_index("s")
        base = wid * tok_per_w

        pltpu.sync_copy(logits_hbm.at[pl.ds(base, tok_per_w)], lbuf)

        iota = lax.iota(jnp.int32, LANES)
        lane_lo = iota < TOPK

        def topk_one(tok):
            ks, vs = [], []
            for g in range(NUM_EXPERTS // LANES):
                x = lbuf[tok, pl.ds(g * LANES, LANES)]
                k_, v_ = plsc.sort_key_val(x, iota + g * LANES,
                                           descending=(g % 2 == 0))
                ks.append(k_)
                vs.append(v_)
            p = jnp.where(lane_lo, ks[0], ks[1])
            pi = jnp.where(lane_lo, vs[0], vs[1])
            q = jnp.where(lane_lo, ks[2], ks[3])
            qi = jnp.where(lane_lo, vs[2], vs[3])
            p, pi = plsc.sort_key_val(p, pi, descending=True)
            q, qi = plsc.sort_key_val(q, qi, descending=False)
            r = jnp.where(lane_lo, p, q)
            ri = jnp.where(lane_lo, pi, qi)
            r, ri = plsc.sort_key_val(r, ri, descending=True)
            e = jnp.where(lane_lo, jnp.exp(r), 0.0)
            w = e / jnp.broadcast_to(jnp.sum(e), (LANES,))
            return w, ri

        @plsc.parallel_loop(0, tok_per_w, unroll=8)
        def body(tok):
            w, ri = topk_one(tok)
            rows = jnp.full((LANES,), tok >> 4, jnp.int32)
            cols = iota + (tok & 15) * TOPK
            plsc.store_scatter(wbuf, [rows, cols], w, mask=lane_lo)
            plsc.store_scatter(ibuf, [rows, cols], ri, mask=lane_lo)

        rows_128 = tok_per_w * TOPK // 128
        pltpu.sync_copy(wbuf, out_w_hbm.at[pl.ds(wid * rows_128, rows_128)])
        pltpu.sync_copy(ibuf, out_i_hbm.at[pl.ds(wid * rows_128, rows_128)])

    return sc_kernel


_SC_KERNEL = _make_sc_kernel()


def kernel(router_logits_fp32, topk_ids, topk_weights):
    w, ids = _SC_KERNEL(router_logits_fp32)
    w = w.reshape(NUM_TOKENS, TOPK).astype(topk_weights.dtype)
    ids = ids.reshape(NUM_TOKENS, TOPK).astype(topk_ids.dtype)
    return (w, ids)
